# baseline (speedup 1.0000x reference)
import functools

import jax
import jax.numpy as jnp
from jax import lax
from jax.experimental import pallas as pl
from jax.experimental.pallas import tpu as pltpu
from jax.experimental.pallas import tpu_sc as plsc

N = 50000
T = 4
F = 16
HG = 64
HT = 128
E = 800000

NC = 2
NS = 16
CH = 128
NCH = 391
E_PAD = NS * NCH * CH
N_PAD = 50048
HN = N_PAD // 2
QN = N_PAD // 4
RPT = N_PAD // NS
RPH = HN // NS
RPQ = QN // NS
WQ = 16
ZR = 256

_mesh = functools.partial(
    plsc.VectorSubcoreMesh, core_axis_name="c", subcore_axis_name="s",
    num_cores=NC, num_subcores=NS)

_sc_params = pltpu.CompilerParams(use_tc_tiling_on_sc=False)


def _fill(ref, rows, value):
  v16 = jnp.full((16,), value, jnp.float32)

  def body(k, carry):
    ref[k] = v16
    return carry

  lax.fori_loop(0, rows, body, 0)


def _fill1(ref, n, value):
  v16 = jnp.full((16,), value, jnp.float32)
  for k in range(n // 16):
    ref[pl.ds(k * 16, 16)] = v16


def _zero_rows(acc, zbuf, base, rows):
  def zcopy(q, carry):
    pltpu.sync_copy(zbuf, acc.at[pl.ds(base + q * ZR, ZR)])
    return carry

  lax.fori_loop(0, rows // ZR, zcopy, 0)
  if rows % ZR:
    pltpu.sync_copy(
        zbuf.at[pl.ds(0, rows % ZR)],
        acc.at[pl.ds(base + (rows // ZR) * ZR, rows % ZR)])


def _mask_idx(src_v, dst_v, node_base):
  neg1 = jnp.full((16,), -1, jnp.int32)

  def body(j, carry):
    for k in range(CH // 16):
      sv = src_v[j, pl.ds(k * 16, 16)]
      dv = dst_v[j, pl.ds(k * 16, 16)]
      off = dv - node_base
      m = (off >= 0) & (off < QN)
      src_v[j, pl.ds(k * 16, 16)] = jnp.where(m, sv, neg1)
      dst_v[j, pl.ds(k * 16, 16)] = jnp.where(m, off, neg1)
    return carry

  lax.fori_loop(0, NCH, body, 0)


def _gidx(src_v, j):
  return plsc.Indices(src_v.at[j], ignored_value=-1)


def _sidx(dst_v, j):
  return plsc.Indices(dst_v.at[j], ignored_value=-1)


def _subpass(mytab, src_v, dst_v, rows_v, acc, sem_g, sem_s):
  pltpu.async_copy(mytab.at[_gidx(src_v, 0)], rows_v.at[0], sem_g)
  pltpu.async_copy(mytab.at[_gidx(src_v, 1)], rows_v.at[1], sem_g)

  def body(j, carry):
    pltpu.make_async_copy(mytab.at[_gidx(src_v, j)],
                          rows_v.at[j % 4], sem_g).wait()
    pltpu.async_copy(rows_v.at[j % 4], acc.at[_sidx(dst_v, j)], sem_s,
                     add=True)
    @pl.when(j + 2 < NCH)
    def _():
      pltpu.async_copy(mytab.at[_gidx(src_v, j + 2)],
                       rows_v.at[(j + 2) % 4], sem_g)
    @pl.when(j >= 1)
    def _():
      pltpu.make_async_copy(rows_v.at[(j - 1) % 4],
                            acc.at[_sidx(dst_v, j - 1)], sem_s).wait()
    return carry

  lax.fori_loop(0, NCH, body, 0)
  pltpu.make_async_copy(rows_v.at[(NCH - 1) % 4],
                        acc.at[_sidx(dst_v, NCH - 1)], sem_s).wait()


def _make_sc_a():
  @functools.partial(
      pl.kernel,
      mesh=_mesh(),
      compiler_params=_sc_params,
      out_type=[
          jax.ShapeDtypeStruct((T, N_PAD, WQ), jnp.float32),
          jax.ShapeDtypeStruct((NC, N_PAD), jnp.float32),
      ],
      scratch_types=[
          pltpu.VMEM((NCH, CH), jnp.int32),
          pltpu.VMEM((NCH, CH), jnp.int32),
          pltpu.VMEM((4, CH, WQ), jnp.float32),
          pltpu.VMEM((ZR, WQ), jnp.float32),
          pltpu.VMEM((ZR,), jnp.float32),
          pltpu.VMEM((CH,), jnp.float32),
          pltpu.VMEM_SHARED((QN, WQ), jnp.float32),
          pltpu.VMEM_SHARED((N_PAD,), jnp.float32),
          pltpu.SemaphoreType.DMA,
          pltpu.SemaphoreType.DMA,
      ],
  )
  def sc_a(tab, src_r, dst_r, out, cnt_out,
           src_v, dst_v, rows_v, zbuf, zbuf1, ones_v, acc, acc1,
           sem_g, sem_s):
    c = lax.axis_index("c")
    s = lax.axis_index("s")
    pltpu.sync_copy(dst_r.at[s], dst_v)
    _fill(zbuf, ZR, 0.0)
    _fill1(zbuf1, ZR, 0.0)
    _fill1(ones_v, CH, 1.0)
    _zero_rows(acc, zbuf, s * RPQ, RPQ)
    _zero_rows(acc1, zbuf1, s * RPT, RPT)
    plsc.subcore_barrier()

    lo = jnp.where(c == 0, 0, NCH // 2)
    hi = jnp.where(c == 0, NCH // 2, NCH)

    def cbody(j, carry):
      pltpu.sync_copy(ones_v, acc1.at[dst_v.at[j]], add=True)
      return carry

    lax.fori_loop(lo, hi, cbody, 0)
    plsc.subcore_barrier()
    pltpu.sync_copy(acc1.at[pl.ds(s * RPT, RPT)],
                    cnt_out.at[c, pl.ds(s * RPT, RPT)])

    def hpass(h, carry):
      pltpu.sync_copy(src_r.at[s], src_v)
      pltpu.sync_copy(dst_r.at[s], dst_v)
      _mask_idx(src_v, dst_v, h * QN)

      def upass(g, carry2):
        q = 2 * g + c
        _subpass(tab.at[q], src_v, dst_v, rows_v, acc, sem_g, sem_s)
        plsc.subcore_barrier()
        pltpu.sync_copy(acc.at[pl.ds(s * RPQ, RPQ)],
                        out.at[q, pl.ds(h * QN + s * RPQ, RPQ)])
        _zero_rows(acc, zbuf, s * RPQ, RPQ)
        plsc.subcore_barrier()
        return carry2

      lax.fori_loop(0, 2, upass, 0)
      return carry

    lax.fori_loop(0, 4, hpass, 0)

  return sc_a


def _make_sc_b():
  @functools.partial(
      pl.kernel,
      mesh=_mesh(),
      compiler_params=_sc_params,
      out_type=jax.ShapeDtypeStruct((T, 4, N_PAD, WQ), jnp.float32),
      scratch_types=[
          pltpu.VMEM((NCH, CH), jnp.int32),
          pltpu.VMEM((NCH, CH), jnp.int32),
          pltpu.VMEM((4, CH, WQ), jnp.float32),
          pltpu.VMEM((ZR, WQ), jnp.float32),
          pltpu.VMEM_SHARED((QN, WQ), jnp.float32),
          pltpu.SemaphoreType.DMA,
          pltpu.SemaphoreType.DMA,
      ],
  )
  def sc_b(hst, src_r, dst_r, out,
           src_v, dst_v, rows_v, zbuf, acc, sem_g, sem_s):
    c = lax.axis_index("c")
    s = lax.axis_index("s")
    _fill(zbuf, ZR, 0.0)
    _zero_rows(acc, zbuf, s * RPQ, RPQ)
    plsc.subcore_barrier()

    def hpass(h, carry):
      pltpu.sync_copy(src_r.at[s], src_v)
      pltpu.sync_copy(dst_r.at[s], dst_v)
      _mask_idx(src_v, dst_v, h * QN)

      def upass(u, carry2):
        t = u // 2
        g = u % 2
        q = 2 * g + c
        _subpass(hst.at[t, q], src_v, dst_v, rows_v, acc, sem_g, sem_s)
        plsc.subcore_barrier()
        pltpu.sync_copy(acc.at[pl.ds(s * RPQ, RPQ)],
                        out.at[t, q, pl.ds(h * QN + s * RPQ, RPQ)])
        _zero_rows(acc, zbuf, s * RPQ, RPQ)
        plsc.subcore_barrier()
        return carry2

      lax.fori_loop(0, 2 * T, upass, 0)
      return carry

    lax.fori_loop(0, 4, hpass, 0)

  return sc_b


@functools.cache
def _sc_a():
  return _make_sc_a()


@functools.cache
def _sc_b():
  return _make_sc_b()


BN = 1000
GRID = N // BN


def _dense0_body(x_ref, a_ref, c_ref, wl_ref, wr_ref, b_ref, g_ref, be_ref,
                 o_ref):
  cnt = c_ref[0] + c_ref[1]
  inv = 1.0 / jnp.maximum(cnt, 1.0)
  wl = wl_ref[...]
  wr = wr_ref[...]
  for t in range(T):
    xt = x_ref[t]
    mean = a_ref[t] * inv
    z = (jnp.dot(mean, wl, preferred_element_type=jnp.float32) + b_ref[...]
         + jnp.dot(xt, wr, preferred_element_type=jnp.float32))
    mu = jnp.mean(z, axis=-1, keepdims=True)
    var = jnp.mean((z - mu) * (z - mu), axis=-1, keepdims=True)
    h = jnp.maximum(
        (z - mu) * lax.rsqrt(var + 1e-5) * g_ref[...] + be_ref[...], 0.0)
    for q in range(4):
      o_ref[t, q] = h[:, q * WQ:(q + 1) * WQ]


def _dense1_body(h_ref, a_ref, c_ref, wl_ref, wr_ref, b_ref, g_ref, be_ref,
                 wih_ref, whh_ref, bih_ref, bhh_ref, wh_ref, bh_ref, o_ref):
  cnt = c_ref[0] + c_ref[1]
  inv = 1.0 / jnp.maximum(cnt, 1.0)
  wl = wl_ref[...]
  wr = wr_ref[...]
  wih = wih_ref[...]
  whh = whh_ref[...]
  bih = bih_ref[...]
  bhh = bhh_ref[...]
  hs = jnp.zeros((BN, HT), jnp.float32)
  for t in range(T):
    ht = jnp.concatenate([h_ref[t, q] for q in range(4)], axis=1)
    at_ = jnp.concatenate([a_ref[t, q] for q in range(4)], axis=1)
    mean = at_ * inv
    z = (jnp.dot(mean, wl, preferred_element_type=jnp.float32) + b_ref[...]
         + jnp.dot(ht, wr, preferred_element_type=jnp.float32))
    mu = jnp.mean(z, axis=-1, keepdims=True)
    var = jnp.mean((z - mu) * (z - mu), axis=-1, keepdims=True)
    hg = jnp.maximum(
        (z - mu) * lax.rsqrt(var + 1e-5) * g_ref[...] + be_ref[...], 0.0)
    gi = jnp.dot(hg, wih, preferred_element_type=jnp.float32) + bih
    gh = jnp.dot(hs, whh, preferred_element_type=jnp.float32) + bhh
    r = jax.nn.sigmoid(gi[:, :HT] + gh[:, :HT])
    zz = jax.nn.sigmoid(gi[:, HT:2 * HT] + gh[:, HT:2 * HT])
    n = jnp.tanh(gi[:, 2 * HT:] + r * gh[:, 2 * HT:])
    hs = (1.0 - zz) * n + zz * hs
  o_ref[...] = jnp.dot(hs, wh_ref[...],
                       preferred_element_type=jnp.float32) + bh_ref[...]


def _full_spec(shape):
  return pl.BlockSpec(shape, lambda i: tuple(0 for _ in shape))


_x_spec = pl.BlockSpec((T, BN, WQ), lambda i: (0, i, 0))
_h_spec = pl.BlockSpec((T, 4, BN, WQ), lambda i: (0, 0, i, 0))
_c_spec = pl.BlockSpec((2, BN, 1), lambda i: (0, i, 0))

_dense0 = pl.pallas_call(
    _dense0_body,
    grid=(GRID,),
    in_specs=[
        _x_spec,
        _x_spec,
        _c_spec,
        _full_spec((F, HG)),
        _full_spec((F, HG)),
        _full_spec((1, HG)),
        _full_spec((1, HG)),
        _full_spec((1, HG)),
    ],
    out_specs=_h_spec,
    out_shape=jax.ShapeDtypeStruct((T, 4, N, WQ), jnp.float32),
)

_dense1 = pl.pallas_call(
    _dense1_body,
    grid=(GRID,),
    in_specs=[
        _h_spec,
        _h_spec,
        _c_spec,
        _full_spec((HG, HG)),
        _full_spec((HG, HG)),
        _full_spec((1, HG)),
        _full_spec((1, HG)),
        _full_spec((1, HG)),
        _full_spec((HG, 3 * HT)),
        _full_spec((HT, 3 * HT)),
        _full_spec((1, 3 * HT)),
        _full_spec((1, 3 * HT)),
        _full_spec((HT, 1)),
        _full_spec((1, 1)),
    ],
    out_specs=pl.BlockSpec((BN, 1), lambda i: (i, 0)),
    out_shape=jax.ShapeDtypeStruct((N, 1), jnp.float32),
)


def kernel(x_seq, edge_index, W_l0, b_l0, W_r0, ln0_g, ln0_b, W_l1, b_l1,
           W_r1, ln1_g, ln1_b, W_ih, W_hh, b_ih, b_hh, W_head, b_head):
  src = edge_index[0]
  dst = edge_index[1]
  pad = E_PAD - E
  src_r = jnp.concatenate(
      [src, jnp.zeros((pad,), jnp.int32)]).reshape(NS, NCH, CH)
  dst_r = jnp.concatenate(
      [dst, jnp.full((pad,), N, jnp.int32)]).reshape(NS, NCH, CH)

  x4 = x_seq[0]

  agg0, cnt2 = _sc_a()(x4, src_r, dst_r)
  cnt2 = cnt2.reshape(NC, N_PAD, 1)

  hst = _dense0(
      x4, agg0, cnt2,
      W_l0.T, W_r0.T,
      b_l0.reshape(1, HG), ln0_g.reshape(1, HG), ln0_b.reshape(1, HG))

  agg1 = _sc_b()(hst, src_r, dst_r)

  y = _dense1(
      hst, agg1, cnt2,
      W_l1.T, W_r1.T,
      b_l1.reshape(1, HG), ln1_g.reshape(1, HG), ln1_b.reshape(1, HG),
      W_ih.T, W_hh.T, b_ih.reshape(1, 3 * HT), b_hh.reshape(1, 3 * HT),
      W_head.T, b_head.reshape(1, 1))
  return y[:, 0]


# speedup vs baseline: 1.1853x; 1.1853x over previous
import functools

import jax
import jax.numpy as jnp
from jax import lax
from jax.experimental import pallas as pl
from jax.experimental.pallas import tpu as pltpu
from jax.experimental.pallas import tpu_sc as plsc

N = 50000
T = 4
F = 16
HG = 64
HT = 128
E = 800000

NC = 2
NS = 16
CH = 128
NCH = 391
E_PAD = NS * NCH * CH
N_PAD = 50048
HN = N_PAD // 2
QN = N_PAD // 4
RPT = N_PAD // NS
RPH = HN // NS
RPQ = QN // NS
WQ = 16
ZR = 256

_mesh = functools.partial(
    plsc.VectorSubcoreMesh, core_axis_name="c", subcore_axis_name="s",
    num_cores=NC, num_subcores=NS)

_sc_params = pltpu.CompilerParams(use_tc_tiling_on_sc=False)


def _fill(ref, rows, value):
  v16 = jnp.full((16,), value, jnp.float32)

  def body(k, carry):
    ref[k] = v16
    return carry

  lax.fori_loop(0, rows, body, 0)


def _fill1(ref, n, value):
  v16 = jnp.full((16,), value, jnp.float32)
  for k in range(n // 16):
    ref[pl.ds(k * 16, 16)] = v16


def _zero_rows(acc, zbuf, base, rows):
  def zcopy(q, carry):
    pltpu.sync_copy(zbuf, acc.at[pl.ds(base + q * ZR, ZR)])
    return carry

  lax.fori_loop(0, rows // ZR, zcopy, 0)
  if rows % ZR:
    pltpu.sync_copy(
        zbuf.at[pl.ds(0, rows % ZR)],
        acc.at[pl.ds(base + (rows // ZR) * ZR, rows % ZR)])


def _mask_idx(src_v, dst_v, node_base):
  neg1 = jnp.full((16,), -1, jnp.int32)

  def body(j, carry):
    for k in range(CH // 16):
      sv = src_v[j, pl.ds(k * 16, 16)]
      dv = dst_v[j, pl.ds(k * 16, 16)]
      off = dv - node_base
      m = (off >= 0) & (off < QN)
      src_v[j, pl.ds(k * 16, 16)] = jnp.where(m, sv, neg1)
      dst_v[j, pl.ds(k * 16, 16)] = jnp.where(m, off, neg1)
    return carry

  lax.fori_loop(0, NCH, body, 0)


def _gidx(src_v, j):
  return plsc.Indices(src_v.at[j], ignored_value=-1)


def _sidx(dst_v, j):
  return plsc.Indices(dst_v.at[j], ignored_value=-1)


def _subpass(mytab, src_v, dst_v, rows_v, acc, sem_g, sem_s):
  del sem_s
  pltpu.async_copy(mytab.at[_gidx(src_v, 0)], rows_v.at[0], sem_g)

  def body(j, carry):
    @pl.when(j + 1 < NCH)
    def _():
      pltpu.async_copy(mytab.at[_gidx(src_v, j + 1)],
                       rows_v.at[(j + 1) % 2], sem_g)
    pltpu.make_async_copy(mytab.at[_gidx(src_v, j)],
                          rows_v.at[j % 2], sem_g).wait()
    pltpu.sync_copy(rows_v.at[j % 2], acc.at[_sidx(dst_v, j)], add=True)
    return carry

  lax.fori_loop(0, NCH, body, 0)


def _make_sc_a():
  @functools.partial(
      pl.kernel,
      mesh=_mesh(),
      compiler_params=_sc_params,
      out_type=[
          jax.ShapeDtypeStruct((T, N_PAD, WQ), jnp.float32),
          jax.ShapeDtypeStruct((NC, N_PAD), jnp.float32),
      ],
      scratch_types=[
          pltpu.VMEM((NCH, CH), jnp.int32),
          pltpu.VMEM((NCH, CH), jnp.int32),
          pltpu.VMEM((4, CH, WQ), jnp.float32),
          pltpu.VMEM((ZR, WQ), jnp.float32),
          pltpu.VMEM((ZR,), jnp.float32),
          pltpu.VMEM((CH,), jnp.float32),
          pltpu.VMEM_SHARED((QN, WQ), jnp.float32),
          pltpu.VMEM_SHARED((N_PAD,), jnp.float32),
          pltpu.SemaphoreType.DMA,
          pltpu.SemaphoreType.DMA,
      ],
  )
  def sc_a(tab, src_r, dst_r, out, cnt_out,
           src_v, dst_v, rows_v, zbuf, zbuf1, ones_v, acc, acc1,
           sem_g, sem_s):
    c = lax.axis_index("c")
    s = lax.axis_index("s")
    pltpu.sync_copy(dst_r.at[s], dst_v)
    _fill(zbuf, ZR, 0.0)
    _fill1(zbuf1, ZR, 0.0)
    _fill1(ones_v, CH, 1.0)
    _zero_rows(acc, zbuf, s * RPQ, RPQ)
    _zero_rows(acc1, zbuf1, s * RPT, RPT)
    plsc.subcore_barrier()

    lo = jnp.where(c == 0, 0, NCH // 2)
    hi = jnp.where(c == 0, NCH // 2, NCH)

    def cbody(j, carry):
      pltpu.sync_copy(ones_v, acc1.at[dst_v.at[j]], add=True)
      return carry

    lax.fori_loop(lo, hi, cbody, 0)
    plsc.subcore_barrier()
    pltpu.sync_copy(acc1.at[pl.ds(s * RPT, RPT)],
                    cnt_out.at[c, pl.ds(s * RPT, RPT)])

    def hpass(h, carry):
      pltpu.sync_copy(src_r.at[s], src_v)
      pltpu.sync_copy(dst_r.at[s], dst_v)
      _mask_idx(src_v, dst_v, h * QN)

      def upass(g, carry2):
        q = 2 * g + c
        _subpass(tab.at[q], src_v, dst_v, rows_v, acc, sem_g, sem_s)
        plsc.subcore_barrier()
        pltpu.sync_copy(acc.at[pl.ds(s * RPQ, RPQ)],
                        out.at[q, pl.ds(h * QN + s * RPQ, RPQ)])
        _zero_rows(acc, zbuf, s * RPQ, RPQ)
        plsc.subcore_barrier()
        return carry2

      lax.fori_loop(0, 2, upass, 0)
      return carry

    lax.fori_loop(0, 4, hpass, 0)

  return sc_a


def _make_sc_b():
  @functools.partial(
      pl.kernel,
      mesh=_mesh(),
      compiler_params=_sc_params,
      out_type=jax.ShapeDtypeStruct((T, 4, N_PAD, WQ), jnp.float32),
      scratch_types=[
          pltpu.VMEM((NCH, CH), jnp.int32),
          pltpu.VMEM((NCH, CH), jnp.int32),
          pltpu.VMEM((4, CH, WQ), jnp.float32),
          pltpu.VMEM((ZR, WQ), jnp.float32),
          pltpu.VMEM_SHARED((QN, WQ), jnp.float32),
          pltpu.SemaphoreType.DMA,
          pltpu.SemaphoreType.DMA,
      ],
  )
  def sc_b(hst, src_r, dst_r, out,
           src_v, dst_v, rows_v, zbuf, acc, sem_g, sem_s):
    c = lax.axis_index("c")
    s = lax.axis_index("s")
    _fill(zbuf, ZR, 0.0)
    _zero_rows(acc, zbuf, s * RPQ, RPQ)
    plsc.subcore_barrier()

    def hpass(h, carry):
      pltpu.sync_copy(src_r.at[s], src_v)
      pltpu.sync_copy(dst_r.at[s], dst_v)
      _mask_idx(src_v, dst_v, h * QN)

      def upass(u, carry2):
        t = u // 2
        g = u % 2
        q = 2 * g + c
        _subpass(hst.at[t, q], src_v, dst_v, rows_v, acc, sem_g, sem_s)
        plsc.subcore_barrier()
        pltpu.sync_copy(acc.at[pl.ds(s * RPQ, RPQ)],
                        out.at[t, q, pl.ds(h * QN + s * RPQ, RPQ)])
        _zero_rows(acc, zbuf, s * RPQ, RPQ)
        plsc.subcore_barrier()
        return carry2

      lax.fori_loop(0, 2 * T, upass, 0)
      return carry

    lax.fori_loop(0, 4, hpass, 0)

  return sc_b


@functools.cache
def _sc_a():
  return _make_sc_a()


@functools.cache
def _sc_b():
  return _make_sc_b()


BN = 1000
GRID = N // BN


def _dense0_body(x_ref, a_ref, c_ref, wl_ref, wr_ref, b_ref, g_ref, be_ref,
                 o_ref):
  cnt = c_ref[0] + c_ref[1]
  inv = 1.0 / jnp.maximum(cnt, 1.0)
  wl = wl_ref[...]
  wr = wr_ref[...]
  for t in range(T):
    xt = x_ref[t]
    mean = a_ref[t] * inv
    z = (jnp.dot(mean, wl, preferred_element_type=jnp.float32) + b_ref[...]
         + jnp.dot(xt, wr, preferred_element_type=jnp.float32))
    mu = jnp.mean(z, axis=-1, keepdims=True)
    var = jnp.mean((z - mu) * (z - mu), axis=-1, keepdims=True)
    h = jnp.maximum(
        (z - mu) * lax.rsqrt(var + 1e-5) * g_ref[...] + be_ref[...], 0.0)
    for q in range(4):
      o_ref[t, q] = h[:, q * WQ:(q + 1) * WQ]


def _dense1_body(h_ref, a_ref, c_ref, wl_ref, wr_ref, b_ref, g_ref, be_ref,
                 wih_ref, whh_ref, bih_ref, bhh_ref, wh_ref, bh_ref, o_ref):
  cnt = c_ref[0] + c_ref[1]
  inv = 1.0 / jnp.maximum(cnt, 1.0)
  wl = wl_ref[...]
  wr = wr_ref[...]
  wih = wih_ref[...]
  whh = whh_ref[...]
  bih = bih_ref[...]
  bhh = bhh_ref[...]
  hs = jnp.zeros((BN, HT), jnp.float32)
  for t in range(T):
    ht = jnp.concatenate([h_ref[t, q] for q in range(4)], axis=1)
    at_ = jnp.concatenate([a_ref[t, q] for q in range(4)], axis=1)
    mean = at_ * inv
    z = (jnp.dot(mean, wl, preferred_element_type=jnp.float32) + b_ref[...]
         + jnp.dot(ht, wr, preferred_element_type=jnp.float32))
    mu = jnp.mean(z, axis=-1, keepdims=True)
    var = jnp.mean((z - mu) * (z - mu), axis=-1, keepdims=True)
    hg = jnp.maximum(
        (z - mu) * lax.rsqrt(var + 1e-5) * g_ref[...] + be_ref[...], 0.0)
    gi = jnp.dot(hg, wih, preferred_element_type=jnp.float32) + bih
    gh = jnp.dot(hs, whh, preferred_element_type=jnp.float32) + bhh
    r = jax.nn.sigmoid(gi[:, :HT] + gh[:, :HT])
    zz = jax.nn.sigmoid(gi[:, HT:2 * HT] + gh[:, HT:2 * HT])
    n = jnp.tanh(gi[:, 2 * HT:] + r * gh[:, 2 * HT:])
    hs = (1.0 - zz) * n + zz * hs
  o_ref[...] = jnp.dot(hs, wh_ref[...],
                       preferred_element_type=jnp.float32) + bh_ref[...]


def _full_spec(shape):
  return pl.BlockSpec(shape, lambda i: tuple(0 for _ in shape))


_x_spec = pl.BlockSpec((T, BN, WQ), lambda i: (0, i, 0))
_h_spec = pl.BlockSpec((T, 4, BN, WQ), lambda i: (0, 0, i, 0))
_c_spec = pl.BlockSpec((2, BN, 1), lambda i: (0, i, 0))

_dense0 = pl.pallas_call(
    _dense0_body,
    grid=(GRID,),
    in_specs=[
        _x_spec,
        _x_spec,
        _c_spec,
        _full_spec((F, HG)),
        _full_spec((F, HG)),
        _full_spec((1, HG)),
        _full_spec((1, HG)),
        _full_spec((1, HG)),
    ],
    out_specs=_h_spec,
    out_shape=jax.ShapeDtypeStruct((T, 4, N, WQ), jnp.float32),
)

_dense1 = pl.pallas_call(
    _dense1_body,
    grid=(GRID,),
    in_specs=[
        _h_spec,
        _h_spec,
        _c_spec,
        _full_spec((HG, HG)),
        _full_spec((HG, HG)),
        _full_spec((1, HG)),
        _full_spec((1, HG)),
        _full_spec((1, HG)),
        _full_spec((HG, 3 * HT)),
        _full_spec((HT, 3 * HT)),
        _full_spec((1, 3 * HT)),
        _full_spec((1, 3 * HT)),
        _full_spec((HT, 1)),
        _full_spec((1, 1)),
    ],
    out_specs=pl.BlockSpec((BN, 1), lambda i: (i, 0)),
    out_shape=jax.ShapeDtypeStruct((N, 1), jnp.float32),
)


def kernel(x_seq, edge_index, W_l0, b_l0, W_r0, ln0_g, ln0_b, W_l1, b_l1,
           W_r1, ln1_g, ln1_b, W_ih, W_hh, b_ih, b_hh, W_head, b_head):
  src = edge_index[0]
  dst = edge_index[1]
  pad = E_PAD - E
  src_r = jnp.concatenate(
      [src, jnp.zeros((pad,), jnp.int32)]).reshape(NS, NCH, CH)
  dst_r = jnp.concatenate(
      [dst, jnp.full((pad,), N, jnp.int32)]).reshape(NS, NCH, CH)

  x4 = x_seq[0]

  agg0, cnt2 = _sc_a()(x4, src_r, dst_r)
  cnt2 = cnt2.reshape(NC, N_PAD, 1)

  hst = _dense0(
      x4, agg0, cnt2,
      W_l0.T, W_r0.T,
      b_l0.reshape(1, HG), ln0_g.reshape(1, HG), ln0_b.reshape(1, HG))

  agg1 = _sc_b()(hst, src_r, dst_r)

  y = _dense1(
      hst, agg1, cnt2,
      W_l1.T, W_r1.T,
      b_l1.reshape(1, HG), ln1_g.reshape(1, HG), ln1_b.reshape(1, HG),
      W_ih.T, W_hh.T, b_ih.reshape(1, 3 * HT), b_hh.reshape(1, 3 * HT),
      W_head.T, b_head.reshape(1, 1))
  return y[:, 0]


# speedup vs baseline: 1.4988x; 1.2645x over previous
import functools

import jax
import jax.numpy as jnp
from jax import lax
from jax.experimental import pallas as pl
from jax.experimental.pallas import tpu as pltpu
from jax.experimental.pallas import tpu_sc as plsc

N = 50000
T = 4
F = 16
HG = 64
HT = 128
E = 800000

NC = 2
NS = 16
CH = 128
NCH = 391
E_PAD = NS * NCH * CH
N_PAD = 50048
HN = N_PAD // 2
QN = N_PAD // 4
RPT = N_PAD // NS
RPH = HN // NS
RPQ = QN // NS
WQ = 16
ZR = 256

_mesh = functools.partial(
    plsc.VectorSubcoreMesh, core_axis_name="c", subcore_axis_name="s",
    num_cores=NC, num_subcores=NS)

_sc_params = pltpu.CompilerParams(use_tc_tiling_on_sc=False)


def _fill(ref, rows, value):
  v16 = jnp.full((16,), value, jnp.float32)

  def body(k, carry):
    ref[k] = v16
    return carry

  lax.fori_loop(0, rows, body, 0)


def _fill1(ref, n, value):
  v16 = jnp.full((16,), value, jnp.float32)
  for k in range(n // 16):
    ref[pl.ds(k * 16, 16)] = v16


def _zero_rows(acc, zbuf, base, rows):
  def zcopy(q, carry):
    pltpu.sync_copy(zbuf, acc.at[pl.ds(base + q * ZR, ZR)])
    return carry

  lax.fori_loop(0, rows // ZR, zcopy, 0)
  if rows % ZR:
    pltpu.sync_copy(
        zbuf.at[pl.ds(0, rows % ZR)],
        acc.at[pl.ds(base + (rows // ZR) * ZR, rows % ZR)])


def _mask_idx(src_v, dst_v, node_base):
  neg1 = jnp.full((16,), -1, jnp.int32)

  def body(j, carry):
    for k in range(CH // 16):
      sv = src_v[j, pl.ds(k * 16, 16)]
      dv = dst_v[j, pl.ds(k * 16, 16)]
      off = dv - node_base
      m = (off >= 0) & (off < QN)
      src_v[j, pl.ds(k * 16, 16)] = jnp.where(m, sv, neg1)
      dst_v[j, pl.ds(k * 16, 16)] = jnp.where(m, off, neg1)
    return carry

  lax.fori_loop(0, NCH, body, 0)


def _gidx(src_v, j):
  return plsc.Indices(src_v.at[j], ignored_value=-1)


def _sidx(dst_v, j):
  return plsc.Indices(dst_v.at[j], ignored_value=-1)


def _subpass(mytab, src_v, dst_v, rows_v, acc, sem_g, sem_s):
  del sem_s
  pltpu.async_copy(mytab.at[_gidx(src_v, 0)], rows_v.at[0], sem_g)
  pltpu.async_copy(mytab.at[_gidx(src_v, 1)], rows_v.at[1], sem_g)
  pltpu.async_copy(mytab.at[_gidx(src_v, 2)], rows_v.at[2], sem_g)

  def body(j, carry):
    pltpu.make_async_copy(mytab.at[_gidx(src_v, j)],
                          rows_v.at[j % 4], sem_g).wait()
    @pl.when(j + 3 < NCH)
    def _():
      pltpu.async_copy(mytab.at[_gidx(src_v, j + 3)],
                       rows_v.at[(j + 3) % 4], sem_g)
    pltpu.sync_copy(rows_v.at[j % 4], acc.at[_sidx(dst_v, j)], add=True)
    return carry

  lax.fori_loop(0, NCH, body, 0)


def _make_sc_a():
  @functools.partial(
      pl.kernel,
      mesh=_mesh(),
      compiler_params=_sc_params,
      out_type=[
          jax.ShapeDtypeStruct((T, N_PAD, WQ), jnp.float32),
          jax.ShapeDtypeStruct((NC, N_PAD), jnp.float32),
      ],
      scratch_types=[
          pltpu.VMEM((NCH, CH), jnp.int32),
          pltpu.VMEM((NCH, CH), jnp.int32),
          pltpu.VMEM((4, CH, WQ), jnp.float32),
          pltpu.VMEM((ZR, WQ), jnp.float32),
          pltpu.VMEM((ZR,), jnp.float32),
          pltpu.VMEM((CH,), jnp.float32),
          pltpu.VMEM_SHARED((QN, WQ), jnp.float32),
          pltpu.VMEM_SHARED((N_PAD,), jnp.float32),
          pltpu.SemaphoreType.DMA,
          pltpu.SemaphoreType.DMA,
      ],
  )
  def sc_a(tab, src_r, dst_r, out, cnt_out,
           src_v, dst_v, rows_v, zbuf, zbuf1, ones_v, acc, acc1,
           sem_g, sem_s):
    c = lax.axis_index("c")
    s = lax.axis_index("s")
    pltpu.sync_copy(dst_r.at[s], dst_v)
    _fill(zbuf, ZR, 0.0)
    _fill1(zbuf1, ZR, 0.0)
    _fill1(ones_v, CH, 1.0)
    _zero_rows(acc, zbuf, s * RPQ, RPQ)
    _zero_rows(acc1, zbuf1, s * RPT, RPT)
    plsc.subcore_barrier()

    lo = jnp.where(c == 0, 0, NCH // 2)
    hi = jnp.where(c == 0, NCH // 2, NCH)

    def cbody(j, carry):
      pltpu.sync_copy(ones_v, acc1.at[dst_v.at[j]], add=True)
      return carry

    lax.fori_loop(lo, hi, cbody, 0)
    plsc.subcore_barrier()
    pltpu.sync_copy(acc1.at[pl.ds(s * RPT, RPT)],
                    cnt_out.at[c, pl.ds(s * RPT, RPT)])

    def hpass(h, carry):
      pltpu.sync_copy(src_r.at[s], src_v)
      pltpu.sync_copy(dst_r.at[s], dst_v)
      _mask_idx(src_v, dst_v, h * QN)

      def upass(g, carry2):
        q = 2 * g + c
        _subpass(tab.at[q], src_v, dst_v, rows_v, acc, sem_g, sem_s)
        plsc.subcore_barrier()
        pltpu.sync_copy(acc.at[pl.ds(s * RPQ, RPQ)],
                        out.at[q, pl.ds(h * QN + s * RPQ, RPQ)])
        _zero_rows(acc, zbuf, s * RPQ, RPQ)
        plsc.subcore_barrier()
        return carry2

      lax.fori_loop(0, 2, upass, 0)
      return carry

    lax.fori_loop(0, 4, hpass, 0)

  return sc_a


def _make_sc_b():
  @functools.partial(
      pl.kernel,
      mesh=_mesh(),
      compiler_params=_sc_params,
      out_type=jax.ShapeDtypeStruct((T, 4, N_PAD, WQ), jnp.float32),
      scratch_types=[
          pltpu.VMEM((NCH, CH), jnp.int32),
          pltpu.VMEM((NCH, CH), jnp.int32),
          pltpu.VMEM((4, CH, WQ), jnp.float32),
          pltpu.VMEM((ZR, WQ), jnp.float32),
          pltpu.VMEM_SHARED((QN, WQ), jnp.float32),
          pltpu.SemaphoreType.DMA,
          pltpu.SemaphoreType.DMA,
      ],
  )
  def sc_b(hst, src_r, dst_r, out,
           src_v, dst_v, rows_v, zbuf, acc, sem_g, sem_s):
    c = lax.axis_index("c")
    s = lax.axis_index("s")
    _fill(zbuf, ZR, 0.0)
    _zero_rows(acc, zbuf, s * RPQ, RPQ)
    plsc.subcore_barrier()

    def hpass(h, carry):
      pltpu.sync_copy(src_r.at[s], src_v)
      pltpu.sync_copy(dst_r.at[s], dst_v)
      _mask_idx(src_v, dst_v, h * QN)

      def upass(u, carry2):
        t = u // 2
        g = u % 2
        q = 2 * g + c
        _subpass(hst.at[t, q], src_v, dst_v, rows_v, acc, sem_g, sem_s)
        plsc.subcore_barrier()
        pltpu.sync_copy(acc.at[pl.ds(s * RPQ, RPQ)],
                        out.at[t, q, pl.ds(h * QN + s * RPQ, RPQ)])
        _zero_rows(acc, zbuf, s * RPQ, RPQ)
        plsc.subcore_barrier()
        return carry2

      lax.fori_loop(0, 2 * T, upass, 0)
      return carry

    lax.fori_loop(0, 4, hpass, 0)

  return sc_b


@functools.cache
def _sc_a():
  return _make_sc_a()


@functools.cache
def _sc_b():
  return _make_sc_b()


BN = 1000
GRID = N // BN


def _dense0_body(x_ref, a_ref, c_ref, wl_ref, wr_ref, b_ref, g_ref, be_ref,
                 o_ref):
  cnt = c_ref[0] + c_ref[1]
  inv = 1.0 / jnp.maximum(cnt, 1.0)
  wl = wl_ref[...]
  wr = wr_ref[...]
  for t in range(T):
    xt = x_ref[t]
    mean = a_ref[t] * inv
    z = (jnp.dot(mean, wl, preferred_element_type=jnp.float32) + b_ref[...]
         + jnp.dot(xt, wr, preferred_element_type=jnp.float32))
    mu = jnp.mean(z, axis=-1, keepdims=True)
    var = jnp.mean((z - mu) * (z - mu), axis=-1, keepdims=True)
    h = jnp.maximum(
        (z - mu) * lax.rsqrt(var + 1e-5) * g_ref[...] + be_ref[...], 0.0)
    for q in range(4):
      o_ref[t, q] = h[:, q * WQ:(q + 1) * WQ]


def _dense1_body(h_ref, a_ref, c_ref, wl_ref, wr_ref, b_ref, g_ref, be_ref,
                 wih_ref, whh_ref, bih_ref, bhh_ref, wh_ref, bh_ref, o_ref):
  cnt = c_ref[0] + c_ref[1]
  inv = 1.0 / jnp.maximum(cnt, 1.0)
  wl = wl_ref[...]
  wr = wr_ref[...]
  wih = wih_ref[...]
  whh = whh_ref[...]
  bih = bih_ref[...]
  bhh = bhh_ref[...]
  hs = jnp.zeros((BN, HT), jnp.float32)
  for t in range(T):
    ht = jnp.concatenate([h_ref[t, q] for q in range(4)], axis=1)
    at_ = jnp.concatenate([a_ref[t, q] for q in range(4)], axis=1)
    mean = at_ * inv
    z = (jnp.dot(mean, wl, preferred_element_type=jnp.float32) + b_ref[...]
         + jnp.dot(ht, wr, preferred_element_type=jnp.float32))
    mu = jnp.mean(z, axis=-1, keepdims=True)
    var = jnp.mean((z - mu) * (z - mu), axis=-1, keepdims=True)
    hg = jnp.maximum(
        (z - mu) * lax.rsqrt(var + 1e-5) * g_ref[...] + be_ref[...], 0.0)
    gi = jnp.dot(hg, wih, preferred_element_type=jnp.float32) + bih
    gh = jnp.dot(hs, whh, preferred_element_type=jnp.float32) + bhh
    r = jax.nn.sigmoid(gi[:, :HT] + gh[:, :HT])
    zz = jax.nn.sigmoid(gi[:, HT:2 * HT] + gh[:, HT:2 * HT])
    n = jnp.tanh(gi[:, 2 * HT:] + r * gh[:, 2 * HT:])
    hs = (1.0 - zz) * n + zz * hs
  o_ref[...] = jnp.dot(hs, wh_ref[...],
                       preferred_element_type=jnp.float32) + bh_ref[...]


def _full_spec(shape):
  return pl.BlockSpec(shape, lambda i: tuple(0 for _ in shape))


_x_spec = pl.BlockSpec((T, BN, WQ), lambda i: (0, i, 0))
_h_spec = pl.BlockSpec((T, 4, BN, WQ), lambda i: (0, 0, i, 0))
_c_spec = pl.BlockSpec((2, BN, 1), lambda i: (0, i, 0))

_dense0 = pl.pallas_call(
    _dense0_body,
    grid=(GRID,),
    in_specs=[
        _x_spec,
        _x_spec,
        _c_spec,
        _full_spec((F, HG)),
        _full_spec((F, HG)),
        _full_spec((1, HG)),
        _full_spec((1, HG)),
        _full_spec((1, HG)),
    ],
    out_specs=_h_spec,
    out_shape=jax.ShapeDtypeStruct((T, 4, N, WQ), jnp.float32),
)

_dense1 = pl.pallas_call(
    _dense1_body,
    grid=(GRID,),
    in_specs=[
        _h_spec,
        _h_spec,
        _c_spec,
        _full_spec((HG, HG)),
        _full_spec((HG, HG)),
        _full_spec((1, HG)),
        _full_spec((1, HG)),
        _full_spec((1, HG)),
        _full_spec((HG, 3 * HT)),
        _full_spec((HT, 3 * HT)),
        _full_spec((1, 3 * HT)),
        _full_spec((1, 3 * HT)),
        _full_spec((HT, 1)),
        _full_spec((1, 1)),
    ],
    out_specs=pl.BlockSpec((BN, 1), lambda i: (i, 0)),
    out_shape=jax.ShapeDtypeStruct((N, 1), jnp.float32),
)


def kernel(x_seq, edge_index, W_l0, b_l0, W_r0, ln0_g, ln0_b, W_l1, b_l1,
           W_r1, ln1_g, ln1_b, W_ih, W_hh, b_ih, b_hh, W_head, b_head):
  src = edge_index[0]
  dst = edge_index[1]
  pad = E_PAD - E
  src_r = jnp.concatenate(
      [src, jnp.zeros((pad,), jnp.int32)]).reshape(NS, NCH, CH)
  dst_r = jnp.concatenate(
      [dst, jnp.full((pad,), N, jnp.int32)]).reshape(NS, NCH, CH)

  x4 = x_seq[0]

  agg0, cnt2 = _sc_a()(x4, src_r, dst_r)
  cnt2 = cnt2.reshape(NC, N_PAD, 1)

  hst = _dense0(
      x4, agg0, cnt2,
      W_l0.T, W_r0.T,
      b_l0.reshape(1, HG), ln0_g.reshape(1, HG), ln0_b.reshape(1, HG))

  agg1 = _sc_b()(hst, src_r, dst_r)

  y = _dense1(
      hst, agg1, cnt2,
      W_l1.T, W_r1.T,
      b_l1.reshape(1, HG), ln1_g.reshape(1, HG), ln1_b.reshape(1, HG),
      W_ih.T, W_hh.T, b_ih.reshape(1, 3 * HT), b_hh.reshape(1, 3 * HT),
      W_head.T, b_head.reshape(1, 1))
  return y[:, 0]


# speedup vs baseline: 1.4993x; 1.0003x over previous
import functools

import jax
import jax.numpy as jnp
from jax import lax
from jax.experimental import pallas as pl
from jax.experimental.pallas import tpu as pltpu
from jax.experimental.pallas import tpu_sc as plsc

N = 50000
T = 4
F = 16
HG = 64
HT = 128
E = 800000

NC = 2
NS = 16
CH = 128
NCH = 391
E_PAD = NS * NCH * CH
N_PAD = 50048
HN = N_PAD // 2
QN = N_PAD // 4
RPT = N_PAD // NS
RPH = HN // NS
RPQ = QN // NS
WQ = 16
NB = 4
ZR = 256

_mesh = functools.partial(
    plsc.VectorSubcoreMesh, core_axis_name="c", subcore_axis_name="s",
    num_cores=NC, num_subcores=NS)

_sc_params = pltpu.CompilerParams(use_tc_tiling_on_sc=False)


def _fill(ref, rows, value):
  v16 = jnp.full((16,), value, jnp.float32)

  def body(k, carry):
    ref[k] = v16
    return carry

  lax.fori_loop(0, rows, body, 0)


def _fill1(ref, n, value):
  v16 = jnp.full((16,), value, jnp.float32)
  for k in range(n // 16):
    ref[pl.ds(k * 16, 16)] = v16


def _zero_rows(acc, zbuf, base, rows):
  def zcopy(q, carry):
    pltpu.sync_copy(zbuf, acc.at[pl.ds(base + q * ZR, ZR)])
    return carry

  lax.fori_loop(0, rows // ZR, zcopy, 0)
  if rows % ZR:
    pltpu.sync_copy(
        zbuf.at[pl.ds(0, rows % ZR)],
        acc.at[pl.ds(base + (rows // ZR) * ZR, rows % ZR)])


def _mask_idx(src_v, dst_v, node_base):
  neg1 = jnp.full((16,), -1, jnp.int32)

  def body(j, carry):
    for k in range(CH // 16):
      sv = src_v[j, pl.ds(k * 16, 16)]
      dv = dst_v[j, pl.ds(k * 16, 16)]
      off = dv - node_base
      m = (off >= 0) & (off < QN)
      src_v[j, pl.ds(k * 16, 16)] = jnp.where(m, sv, neg1)
      dst_v[j, pl.ds(k * 16, 16)] = jnp.where(m, off, neg1)
    return carry

  lax.fori_loop(0, NCH, body, 0)


def _gidx(src_v, j):
  return plsc.Indices(src_v.at[j], ignored_value=-1)


def _sidx(dst_v, j):
  return plsc.Indices(dst_v.at[j], ignored_value=-1)


def _subpass(mytab, src_v, dst_v, rows_v, acc, sem_g, sem_s):
  del sem_s
  for p in range(NB - 1):
    pltpu.async_copy(mytab.at[_gidx(src_v, p)], rows_v.at[p], sem_g)

  def body(j, carry):
    pltpu.make_async_copy(mytab.at[_gidx(src_v, j)],
                          rows_v.at[j % NB], sem_g).wait()
    @pl.when(j + NB - 1 < NCH)
    def _():
      pltpu.async_copy(mytab.at[_gidx(src_v, j + NB - 1)],
                       rows_v.at[(j + NB - 1) % NB], sem_g)
    pltpu.sync_copy(rows_v.at[j % NB], acc.at[_sidx(dst_v, j)], add=True)
    return carry

  lax.fori_loop(0, NCH, body, 0)


def _make_sc_a():
  @functools.partial(
      pl.kernel,
      mesh=_mesh(),
      compiler_params=_sc_params,
      out_type=[
          jax.ShapeDtypeStruct((T, N_PAD, WQ), jnp.float32),
          jax.ShapeDtypeStruct((NC, N_PAD), jnp.float32),
      ],
      scratch_types=[
          pltpu.VMEM((NCH, CH), jnp.int32),
          pltpu.VMEM((NCH, CH), jnp.int32),
          pltpu.VMEM((NB, CH, WQ), jnp.float32),
          pltpu.VMEM((ZR, WQ), jnp.float32),
          pltpu.VMEM((ZR,), jnp.float32),
          pltpu.VMEM((CH,), jnp.float32),
          pltpu.VMEM_SHARED((QN, WQ), jnp.float32),
          pltpu.VMEM_SHARED((N_PAD,), jnp.float32),
          pltpu.SemaphoreType.DMA,
          pltpu.SemaphoreType.DMA,
      ],
  )
  def sc_a(tab, src_r, dst_r, out, cnt_out,
           src_v, dst_v, rows_v, zbuf, zbuf1, ones_v, acc, acc1,
           sem_g, sem_s):
    c = lax.axis_index("c")
    s = lax.axis_index("s")
    pltpu.sync_copy(dst_r.at[s], dst_v)
    _fill(zbuf, ZR, 0.0)
    _fill1(zbuf1, ZR, 0.0)
    _fill1(ones_v, CH, 1.0)
    _zero_rows(acc, zbuf, s * RPQ, RPQ)
    _zero_rows(acc1, zbuf1, s * RPT, RPT)
    plsc.subcore_barrier()

    lo = jnp.where(c == 0, 0, NCH // 2)
    hi = jnp.where(c == 0, NCH // 2, NCH)

    def cbody(j, carry):
      pltpu.sync_copy(ones_v, acc1.at[dst_v.at[j]], add=True)
      return carry

    lax.fori_loop(lo, hi, cbody, 0)
    plsc.subcore_barrier()
    pltpu.sync_copy(acc1.at[pl.ds(s * RPT, RPT)],
                    cnt_out.at[c, pl.ds(s * RPT, RPT)])

    def hpass(h, carry):
      pltpu.sync_copy(src_r.at[s], src_v)
      pltpu.sync_copy(dst_r.at[s], dst_v)
      _mask_idx(src_v, dst_v, h * QN)

      def upass(g, carry2):
        q = 2 * g + c
        _subpass(tab.at[q], src_v, dst_v, rows_v, acc, sem_g, sem_s)
        plsc.subcore_barrier()
        pltpu.sync_copy(acc.at[pl.ds(s * RPQ, RPQ)],
                        out.at[q, pl.ds(h * QN + s * RPQ, RPQ)])
        _zero_rows(acc, zbuf, s * RPQ, RPQ)
        plsc.subcore_barrier()
        return carry2

      lax.fori_loop(0, 2, upass, 0)
      return carry

    lax.fori_loop(0, 4, hpass, 0)

  return sc_a


def _make_sc_b():
  @functools.partial(
      pl.kernel,
      mesh=_mesh(),
      compiler_params=_sc_params,
      out_type=jax.ShapeDtypeStruct((T, 4, N_PAD, WQ), jnp.float32),
      scratch_types=[
          pltpu.VMEM((NCH, CH), jnp.int32),
          pltpu.VMEM((NCH, CH), jnp.int32),
          pltpu.VMEM((NB, CH, WQ), jnp.float32),
          pltpu.VMEM((ZR, WQ), jnp.float32),
          pltpu.VMEM_SHARED((QN, WQ), jnp.float32),
          pltpu.SemaphoreType.DMA,
          pltpu.SemaphoreType.DMA,
      ],
  )
  def sc_b(hst, src_r, dst_r, out,
           src_v, dst_v, rows_v, zbuf, acc, sem_g, sem_s):
    c = lax.axis_index("c")
    s = lax.axis_index("s")
    _fill(zbuf, ZR, 0.0)
    _zero_rows(acc, zbuf, s * RPQ, RPQ)
    plsc.subcore_barrier()

    def hpass(h, carry):
      pltpu.sync_copy(src_r.at[s], src_v)
      pltpu.sync_copy(dst_r.at[s], dst_v)
      _mask_idx(src_v, dst_v, h * QN)

      def upass(u, carry2):
        t = u // 2
        g = u % 2
        q = 2 * g + c
        _subpass(hst.at[t, q], src_v, dst_v, rows_v, acc, sem_g, sem_s)
        plsc.subcore_barrier()
        pltpu.sync_copy(acc.at[pl.ds(s * RPQ, RPQ)],
                        out.at[t, q, pl.ds(h * QN + s * RPQ, RPQ)])
        _zero_rows(acc, zbuf, s * RPQ, RPQ)
        plsc.subcore_barrier()
        return carry2

      lax.fori_loop(0, 2 * T, upass, 0)
      return carry

    lax.fori_loop(0, 4, hpass, 0)

  return sc_b


@functools.cache
def _sc_a():
  return _make_sc_a()


@functools.cache
def _sc_b():
  return _make_sc_b()


BN = 1000
GRID = N // BN


def _dense0_body(x_ref, a_ref, c_ref, wl_ref, wr_ref, b_ref, g_ref, be_ref,
                 o_ref):
  cnt = c_ref[0] + c_ref[1]
  inv = 1.0 / jnp.maximum(cnt, 1.0)
  wl = wl_ref[...]
  wr = wr_ref[...]
  for t in range(T):
    xt = x_ref[t]
    mean = a_ref[t] * inv
    z = (jnp.dot(mean, wl, preferred_element_type=jnp.float32) + b_ref[...]
         + jnp.dot(xt, wr, preferred_element_type=jnp.float32))
    mu = jnp.mean(z, axis=-1, keepdims=True)
    var = jnp.mean((z - mu) * (z - mu), axis=-1, keepdims=True)
    h = jnp.maximum(
        (z - mu) * lax.rsqrt(var + 1e-5) * g_ref[...] + be_ref[...], 0.0)
    for q in range(4):
      o_ref[t, q] = h[:, q * WQ:(q + 1) * WQ]


def _dense1_body(h_ref, a_ref, c_ref, wl_ref, wr_ref, b_ref, g_ref, be_ref,
                 wih_ref, whh_ref, bih_ref, bhh_ref, wh_ref, bh_ref, o_ref):
  cnt = c_ref[0] + c_ref[1]
  inv = 1.0 / jnp.maximum(cnt, 1.0)
  wl = wl_ref[...]
  wr = wr_ref[...]
  wih = wih_ref[...]
  whh = whh_ref[...]
  bih = bih_ref[...]
  bhh = bhh_ref[...]
  hs = jnp.zeros((BN, HT), jnp.float32)
  for t in range(T):
    ht = jnp.concatenate([h_ref[t, q] for q in range(4)], axis=1)
    at_ = jnp.concatenate([a_ref[t, q] for q in range(4)], axis=1)
    mean = at_ * inv
    z = (jnp.dot(mean, wl, preferred_element_type=jnp.float32) + b_ref[...]
         + jnp.dot(ht, wr, preferred_element_type=jnp.float32))
    mu = jnp.mean(z, axis=-1, keepdims=True)
    var = jnp.mean((z - mu) * (z - mu), axis=-1, keepdims=True)
    hg = jnp.maximum(
        (z - mu) * lax.rsqrt(var + 1e-5) * g_ref[...] + be_ref[...], 0.0)
    gi = jnp.dot(hg, wih, preferred_element_type=jnp.float32) + bih
    gh = jnp.dot(hs, whh, preferred_element_type=jnp.float32) + bhh
    r = jax.nn.sigmoid(gi[:, :HT] + gh[:, :HT])
    zz = jax.nn.sigmoid(gi[:, HT:2 * HT] + gh[:, HT:2 * HT])
    n = jnp.tanh(gi[:, 2 * HT:] + r * gh[:, 2 * HT:])
    hs = (1.0 - zz) * n + zz * hs
  o_ref[...] = jnp.dot(hs, wh_ref[...],
                       preferred_element_type=jnp.float32) + bh_ref[...]


def _full_spec(shape):
  return pl.BlockSpec(shape, lambda i: tuple(0 for _ in shape))


_x_spec = pl.BlockSpec((T, BN, WQ), lambda i: (0, i, 0))
_h_spec = pl.BlockSpec((T, 4, BN, WQ), lambda i: (0, 0, i, 0))
_c_spec = pl.BlockSpec((2, BN, 1), lambda i: (0, i, 0))

_dense0 = pl.pallas_call(
    _dense0_body,
    grid=(GRID,),
    in_specs=[
        _x_spec,
        _x_spec,
        _c_spec,
        _full_spec((F, HG)),
        _full_spec((F, HG)),
        _full_spec((1, HG)),
        _full_spec((1, HG)),
        _full_spec((1, HG)),
    ],
    out_specs=_h_spec,
    out_shape=jax.ShapeDtypeStruct((T, 4, N, WQ), jnp.float32),
)

_dense1 = pl.pallas_call(
    _dense1_body,
    grid=(GRID,),
    in_specs=[
        _h_spec,
        _h_spec,
        _c_spec,
        _full_spec((HG, HG)),
        _full_spec((HG, HG)),
        _full_spec((1, HG)),
        _full_spec((1, HG)),
        _full_spec((1, HG)),
        _full_spec((HG, 3 * HT)),
        _full_spec((HT, 3 * HT)),
        _full_spec((1, 3 * HT)),
        _full_spec((1, 3 * HT)),
        _full_spec((HT, 1)),
        _full_spec((1, 1)),
    ],
    out_specs=pl.BlockSpec((BN, 1), lambda i: (i, 0)),
    out_shape=jax.ShapeDtypeStruct((N, 1), jnp.float32),
)


def kernel(x_seq, edge_index, W_l0, b_l0, W_r0, ln0_g, ln0_b, W_l1, b_l1,
           W_r1, ln1_g, ln1_b, W_ih, W_hh, b_ih, b_hh, W_head, b_head):
  src = edge_index[0]
  dst = edge_index[1]
  pad = E_PAD - E
  src_r = jnp.concatenate(
      [src, jnp.zeros((pad,), jnp.int32)]).reshape(NS, NCH, CH)
  dst_r = jnp.concatenate(
      [dst, jnp.full((pad,), N, jnp.int32)]).reshape(NS, NCH, CH)

  x4 = x_seq[0]

  agg0, cnt2 = _sc_a()(x4, src_r, dst_r)
  cnt2 = cnt2.reshape(NC, N_PAD, 1)

  hst = _dense0(
      x4, agg0, cnt2,
      W_l0.T, W_r0.T,
      b_l0.reshape(1, HG), ln0_g.reshape(1, HG), ln0_b.reshape(1, HG))

  agg1 = _sc_b()(hst, src_r, dst_r)

  y = _dense1(
      hst, agg1, cnt2,
      W_l1.T, W_r1.T,
      b_l1.reshape(1, HG), ln1_g.reshape(1, HG), ln1_b.reshape(1, HG),
      W_ih.T, W_hh.T, b_ih.reshape(1, 3 * HT), b_hh.reshape(1, 3 * HT),
      W_head.T, b_head.reshape(1, 1))
  return y[:, 0]


# speedup vs baseline: 1.8170x; 1.2119x over previous
import functools

import jax
import jax.numpy as jnp
from jax import lax
from jax.experimental import pallas as pl
from jax.experimental.pallas import tpu as pltpu
from jax.experimental.pallas import tpu_sc as plsc

N = 50000
T = 4
F = 16
HG = 64
HT = 128
E = 800000

NC = 2
NS = 16
CH = 128
NCH = 391
E_PAD = NS * NCH * CH
N_PAD = 50048
NSP = 3
SR = 16768
NOUT = NSP * SR
RPT = N_PAD // NS
RPS = SR // NS
WQ = 16
NB = 4
ZR = 256

_mesh = functools.partial(
    plsc.VectorSubcoreMesh, core_axis_name="c", subcore_axis_name="s",
    num_cores=NC, num_subcores=NS)

_sc_params = pltpu.CompilerParams(use_tc_tiling_on_sc=False)


def _fill(ref, rows, value):
  v16 = jnp.full((16,), value, jnp.float32)

  def body(k, carry):
    ref[k] = v16
    return carry

  lax.fori_loop(0, rows, body, 0)


def _fill1(ref, n, value):
  v16 = jnp.full((16,), value, jnp.float32)
  for k in range(n // 16):
    ref[pl.ds(k * 16, 16)] = v16


def _zero_rows(acc, zbuf, base, rows):
  def zcopy(q, carry):
    pltpu.sync_copy(zbuf, acc.at[pl.ds(base + q * ZR, ZR)])
    return carry

  lax.fori_loop(0, rows // ZR, zcopy, 0)
  if rows % ZR:
    pltpu.sync_copy(
        zbuf.at[pl.ds(0, rows % ZR)],
        acc.at[pl.ds(base + (rows // ZR) * ZR, rows % ZR)])


def _mask_idx(src_v, dst_v, node_base, nrows):
  neg1 = jnp.full((16,), -1, jnp.int32)

  def body(j, carry):
    for k in range(CH // 16):
      sv = src_v[j, pl.ds(k * 16, 16)]
      dv = dst_v[j, pl.ds(k * 16, 16)]
      off = dv - node_base
      m = (off >= 0) & (off < nrows)
      src_v[j, pl.ds(k * 16, 16)] = jnp.where(m, sv, neg1)
      dst_v[j, pl.ds(k * 16, 16)] = jnp.where(m, off, neg1)
    return carry

  lax.fori_loop(0, NCH, body, 0)


def _gidx(src_v, j):
  return plsc.Indices(src_v.at[j], ignored_value=-1)


def _sidx(dst_v, j):
  return plsc.Indices(dst_v.at[j], ignored_value=-1)


def _subpass(mytab, src_v, dst_v, rows_v, acc, sem_g, sem_s):
  del sem_s
  for p in range(NB - 1):
    pltpu.async_copy(mytab.at[_gidx(src_v, p)], rows_v.at[p], sem_g)

  def body(j, carry):
    pltpu.make_async_copy(mytab.at[_gidx(src_v, j)],
                          rows_v.at[j % NB], sem_g).wait()
    @pl.when(j + NB - 1 < NCH)
    def _():
      pltpu.async_copy(mytab.at[_gidx(src_v, j + NB - 1)],
                       rows_v.at[(j + NB - 1) % NB], sem_g)
    pltpu.sync_copy(rows_v.at[j % NB], acc.at[_sidx(dst_v, j)], add=True)
    return carry

  lax.fori_loop(0, NCH, body, 0)


def _make_sc_a():
  @functools.partial(
      pl.kernel,
      mesh=_mesh(),
      compiler_params=_sc_params,
      out_type=[
          jax.ShapeDtypeStruct((T, NOUT, WQ), jnp.float32),
          jax.ShapeDtypeStruct((NC, NOUT), jnp.float32),
      ],
      scratch_types=[
          pltpu.VMEM((NCH, CH), jnp.int32),
          pltpu.VMEM((NCH, CH), jnp.int32),
          pltpu.VMEM((NB, CH, WQ), jnp.float32),
          pltpu.VMEM((ZR, WQ), jnp.float32),
          pltpu.VMEM((ZR,), jnp.float32),
          pltpu.VMEM((CH,), jnp.float32),
          pltpu.VMEM_SHARED((SR, WQ), jnp.float32),
          pltpu.VMEM_SHARED((SR,), jnp.float32),
          pltpu.SemaphoreType.DMA,
          pltpu.SemaphoreType.DMA,
      ],
  )
  def sc_a(tab, src_r, dst_r, out, cnt_out,
           src_v, dst_v, rows_v, zbuf, zbuf1, ones_v, acc, acc1,
           sem_g, sem_s):
    c = lax.axis_index("c")
    s = lax.axis_index("s")
    _fill(zbuf, ZR, 0.0)
    _fill1(zbuf1, ZR, 0.0)
    _fill1(ones_v, CH, 1.0)
    _zero_rows(acc, zbuf, s * RPS, RPS)
    _zero_rows(acc1, zbuf1, s * RPS, RPS)
    plsc.subcore_barrier()

    lo = jnp.where(c == 0, 0, NCH // 2)
    hi = jnp.where(c == 0, NCH // 2, NCH)

    def hpass(h, carry):
      pltpu.sync_copy(src_r.at[s], src_v)
      pltpu.sync_copy(dst_r.at[s], dst_v)
      _mask_idx(src_v, dst_v, h * SR, SR)

      def cbody(j, carry2):
        pltpu.sync_copy(ones_v, acc1.at[_sidx(dst_v, j)], add=True)
        return carry2

      lax.fori_loop(lo, hi, cbody, 0)

      def upass(g, carry2):
        q = 2 * g + c
        _subpass(tab.at[q], src_v, dst_v, rows_v, acc, sem_g, sem_s)
        plsc.subcore_barrier()
        pltpu.sync_copy(acc.at[pl.ds(s * RPS, RPS)],
                        out.at[q, pl.ds(h * SR + s * RPS, RPS)])
        _zero_rows(acc, zbuf, s * RPS, RPS)
        plsc.subcore_barrier()
        return carry2

      lax.fori_loop(0, 2, upass, 0)
      pltpu.sync_copy(acc1.at[pl.ds(s * RPS, RPS)],
                      cnt_out.at[c, pl.ds(h * SR + s * RPS, RPS)])
      _zero_rows(acc1, zbuf1, s * RPS, RPS)
      plsc.subcore_barrier()
      return carry

    lax.fori_loop(0, NSP, hpass, 0)

  return sc_a


def _make_sc_b():
  @functools.partial(
      pl.kernel,
      mesh=_mesh(),
      compiler_params=_sc_params,
      out_type=jax.ShapeDtypeStruct((T, 4, NOUT, WQ), jnp.float32),
      scratch_types=[
          pltpu.VMEM((NCH, CH), jnp.int32),
          pltpu.VMEM((NCH, CH), jnp.int32),
          pltpu.VMEM((NB, CH, WQ), jnp.float32),
          pltpu.VMEM((ZR, WQ), jnp.float32),
          pltpu.VMEM_SHARED((SR, WQ), jnp.float32),
          pltpu.SemaphoreType.DMA,
          pltpu.SemaphoreType.DMA,
      ],
  )
  def sc_b(hst, src_r, dst_r, out,
           src_v, dst_v, rows_v, zbuf, acc, sem_g, sem_s):
    c = lax.axis_index("c")
    s = lax.axis_index("s")
    _fill(zbuf, ZR, 0.0)
    _zero_rows(acc, zbuf, s * RPS, RPS)
    plsc.subcore_barrier()

    def hpass(h, carry):
      pltpu.sync_copy(src_r.at[s], src_v)
      pltpu.sync_copy(dst_r.at[s], dst_v)
      _mask_idx(src_v, dst_v, h * SR, SR)

      def upass(u, carry2):
        t = u // 2
        g = u % 2
        q = 2 * g + c
        _subpass(hst.at[t, q], src_v, dst_v, rows_v, acc, sem_g, sem_s)
        plsc.subcore_barrier()
        pltpu.sync_copy(acc.at[pl.ds(s * RPS, RPS)],
                        out.at[t, q, pl.ds(h * SR + s * RPS, RPS)])
        _zero_rows(acc, zbuf, s * RPS, RPS)
        plsc.subcore_barrier()
        return carry2

      lax.fori_loop(0, 2 * T, upass, 0)
      return carry

    lax.fori_loop(0, NSP, hpass, 0)

  return sc_b


@functools.cache
def _sc_a():
  return _make_sc_a()


@functools.cache
def _sc_b():
  return _make_sc_b()


BN = 1000
GRID = N // BN


def _dense0_body(x_ref, a_ref, c_ref, wl_ref, wr_ref, b_ref, g_ref, be_ref,
                 o_ref):
  cnt = c_ref[0] + c_ref[1]
  inv = 1.0 / jnp.maximum(cnt, 1.0)
  wl = wl_ref[...]
  wr = wr_ref[...]
  for t in range(T):
    xt = x_ref[t]
    mean = a_ref[t] * inv
    z = (jnp.dot(mean, wl, preferred_element_type=jnp.float32) + b_ref[...]
         + jnp.dot(xt, wr, preferred_element_type=jnp.float32))
    mu = jnp.mean(z, axis=-1, keepdims=True)
    var = jnp.mean((z - mu) * (z - mu), axis=-1, keepdims=True)
    h = jnp.maximum(
        (z - mu) * lax.rsqrt(var + 1e-5) * g_ref[...] + be_ref[...], 0.0)
    for q in range(4):
      o_ref[t, q] = h[:, q * WQ:(q + 1) * WQ]


def _dense1_body(h_ref, a_ref, c_ref, wl_ref, wr_ref, b_ref, g_ref, be_ref,
                 wih_ref, whh_ref, bih_ref, bhh_ref, wh_ref, bh_ref, o_ref):
  cnt = c_ref[0] + c_ref[1]
  inv = 1.0 / jnp.maximum(cnt, 1.0)
  wl = wl_ref[...]
  wr = wr_ref[...]
  wih = wih_ref[...]
  whh = whh_ref[...]
  bih = bih_ref[...]
  bhh = bhh_ref[...]
  hs = jnp.zeros((BN, HT), jnp.float32)
  for t in range(T):
    ht = jnp.concatenate([h_ref[t, q] for q in range(4)], axis=1)
    at_ = jnp.concatenate([a_ref[t, q] for q in range(4)], axis=1)
    mean = at_ * inv
    z = (jnp.dot(mean, wl, preferred_element_type=jnp.float32) + b_ref[...]
         + jnp.dot(ht, wr, preferred_element_type=jnp.float32))
    mu = jnp.mean(z, axis=-1, keepdims=True)
    var = jnp.mean((z - mu) * (z - mu), axis=-1, keepdims=True)
    hg = jnp.maximum(
        (z - mu) * lax.rsqrt(var + 1e-5) * g_ref[...] + be_ref[...], 0.0)
    gi = jnp.dot(hg, wih, preferred_element_type=jnp.float32) + bih
    gh = jnp.dot(hs, whh, preferred_element_type=jnp.float32) + bhh
    r = jax.nn.sigmoid(gi[:, :HT] + gh[:, :HT])
    zz = jax.nn.sigmoid(gi[:, HT:2 * HT] + gh[:, HT:2 * HT])
    n = jnp.tanh(gi[:, 2 * HT:] + r * gh[:, 2 * HT:])
    hs = (1.0 - zz) * n + zz * hs
  o_ref[...] = jnp.dot(hs, wh_ref[...],
                       preferred_element_type=jnp.float32) + bh_ref[...]


def _full_spec(shape):
  return pl.BlockSpec(shape, lambda i: tuple(0 for _ in shape))


_x_spec = pl.BlockSpec((T, BN, WQ), lambda i: (0, i, 0))
_h_spec = pl.BlockSpec((T, 4, BN, WQ), lambda i: (0, 0, i, 0))
_c_spec = pl.BlockSpec((2, BN, 1), lambda i: (0, i, 0))

_dense0 = pl.pallas_call(
    _dense0_body,
    grid=(GRID,),
    in_specs=[
        _x_spec,
        _x_spec,
        _c_spec,
        _full_spec((F, HG)),
        _full_spec((F, HG)),
        _full_spec((1, HG)),
        _full_spec((1, HG)),
        _full_spec((1, HG)),
    ],
    out_specs=_h_spec,
    out_shape=jax.ShapeDtypeStruct((T, 4, N, WQ), jnp.float32),
)

_dense1 = pl.pallas_call(
    _dense1_body,
    grid=(GRID,),
    in_specs=[
        _h_spec,
        _h_spec,
        _c_spec,
        _full_spec((HG, HG)),
        _full_spec((HG, HG)),
        _full_spec((1, HG)),
        _full_spec((1, HG)),
        _full_spec((1, HG)),
        _full_spec((HG, 3 * HT)),
        _full_spec((HT, 3 * HT)),
        _full_spec((1, 3 * HT)),
        _full_spec((1, 3 * HT)),
        _full_spec((HT, 1)),
        _full_spec((1, 1)),
    ],
    out_specs=pl.BlockSpec((BN, 1), lambda i: (i, 0)),
    out_shape=jax.ShapeDtypeStruct((N, 1), jnp.float32),
)


def kernel(x_seq, edge_index, W_l0, b_l0, W_r0, ln0_g, ln0_b, W_l1, b_l1,
           W_r1, ln1_g, ln1_b, W_ih, W_hh, b_ih, b_hh, W_head, b_head):
  src = edge_index[0]
  dst = edge_index[1]
  pad = E_PAD - E
  src_r = jnp.concatenate(
      [src, jnp.zeros((pad,), jnp.int32)]).reshape(NS, NCH, CH)
  dst_r = jnp.concatenate(
      [dst, jnp.full((pad,), N, jnp.int32)]).reshape(NS, NCH, CH)

  x4 = x_seq[0]

  agg0, cnt2 = _sc_a()(x4, src_r, dst_r)
  cnt2 = cnt2.reshape(NC, NOUT, 1)

  hst = _dense0(
      x4, agg0, cnt2,
      W_l0.T, W_r0.T,
      b_l0.reshape(1, HG), ln0_g.reshape(1, HG), ln0_b.reshape(1, HG))

  agg1 = _sc_b()(hst, src_r, dst_r)

  y = _dense1(
      hst, agg1, cnt2,
      W_l1.T, W_r1.T,
      b_l1.reshape(1, HG), ln1_g.reshape(1, HG), ln1_b.reshape(1, HG),
      W_ih.T, W_hh.T, b_ih.reshape(1, 3 * HT), b_hh.reshape(1, 3 * HT),
      W_head.T, b_head.reshape(1, 1))
  return y[:, 0]
